# Initial kernel scaffold; baseline (speedup 1.0000x reference)
import jax
import jax.numpy as jnp
from jax.experimental import pallas as pl


def kernel(word, pos1, pos2, word_table, pos1_table, pos2_table):
    raise NotImplementedError("write your pallas kernel here")


# speedup vs baseline: 5.3225x; 5.3225x over previous
import functools

import jax
import jax.numpy as jnp
from jax import lax
from jax.experimental import pallas as pl
from jax.experimental.pallas import tpu as pltpu
from jax.experimental.pallas import tpu_sc as plsc

NC = 2
NS = 16
LN = 16
NW = NC * NS

SUB = 128
CHB = 4
CH = SUB * CHB


def _build(N, D, WD, WDP, PD, PV, per_w, n_chunks):
  mesh = plsc.VectorSubcoreMesh(
      core_axis_name="c", subcore_axis_name="s", num_cores=NC,
      num_subcores=NS)

  @functools.partial(
      pl.kernel,
      out_type=jax.ShapeDtypeStruct((N * D,), jnp.float32),
      mesh=mesh,
      compiler_params=pltpu.CompilerParams(
          needs_layout_passes=False, use_tc_tiling_on_sc=False),
      scratch_types=[
          pltpu.VMEM((CHB, SUB), jnp.int32),
          pltpu.VMEM((CH,), jnp.int32),
          pltpu.VMEM((CH,), jnp.int32),
          pltpu.VMEM((CH, WDP), jnp.float32),
          pltpu.VMEM((CH * D,), jnp.float32),
          pltpu.VMEM((PV * PD,), jnp.float32),
          pltpu.VMEM((PV * PD,), jnp.float32),
          pltpu.SemaphoreType.DMA,
      ],
  )
  def k(wf_h, p1_h, p2_h, wt_h, p1t_h, p2t_h, out_h,
        widx, p1i, p2i, wrows, outv, p1t, p2t, gsem):
    wid = lax.axis_index("s") * NC + lax.axis_index("c")
    pltpu.sync_copy(p1t_h, p1t)
    pltpu.sync_copy(p2t_h, p2t)
    lane = lax.iota(jnp.int32, 16)

    @pl.loop(0, n_chunks)
    def _chunk(c):
      row0 = wid * per_w + c * CH
      for b in range(CHB):
        pltpu.sync_copy(wf_h.at[pl.ds(row0 + b * SUB, SUB)], widx.at[b])
      pltpu.sync_copy(p1_h.at[pl.ds(row0, CH)], p1i)
      pltpu.sync_copy(p2_h.at[pl.ds(row0, CH)], p2i)
      cps = [
          pltpu.async_copy(
              wt_h.at[widx.at[b]],
              wrows.at[pl.ds(b * SUB, SUB), :], gsem)
          for b in range(CHB)
      ]

      @pl.loop(0, CH // LN)
      def _pos(g):
        rowsD = (g * LN + lane) * D
        i1 = p1i[pl.ds(g * LN, LN)] * PD
        i2 = p2i[pl.ds(g * LN, LN)] * PD
        for j in range(PD):
          v1 = plsc.load_gather(p1t, [i1 + j])
          plsc.store_scatter(outv, [rowsD + (WD + j)], v1)
          v2 = plsc.load_gather(p2t, [i2 + j])
          plsc.store_scatter(outv, [rowsD + (WD + PD + j)], v2)

      for cp in cps:
        cp.wait()

      @pl.loop(0, CH)
      def _merge(r):
        dst0 = r * D + lane
        for j in (0, LN, 2 * LN, WD - LN):
          v = wrows[r, pl.ds(j, LN)]
          plsc.store_scatter(outv, [dst0 + j], v)

      pltpu.sync_copy(outv, out_h.at[pl.ds(row0 * D, CH * D)])

  return k


def kernel(word, pos1, pos2, word_table, pos1_table, pos2_table):
  B, L = word.shape
  V, WD = word_table.shape
  PV, PD = pos1_table.shape
  D = WD + 2 * PD
  N = B * L
  assert N % (NW * CH) == 0
  per_w = N // NW
  n_chunks = per_w // CH

  wf = word.reshape(N).astype(jnp.int32)
  p1f = pos1.reshape(N).astype(jnp.int32)
  p2f = pos2.reshape(N).astype(jnp.int32)
  p1t = pos1_table.reshape(PV * PD)
  p2t = pos2_table.reshape(PV * PD)

  WDP = 64
  wt_pad = jnp.pad(word_table, ((0, 0), (0, WDP - WD)))

  k = _build(N, D, WD, WDP, PD, PV, per_w, n_chunks)
  out = k(wf, p1f, p2f, wt_pad, p1t, p2t)
  return out.reshape(B, L, D)


# speedup vs baseline: 5.3703x; 1.0090x over previous
import functools

import jax
import jax.numpy as jnp
from jax import lax
from jax.experimental import pallas as pl
from jax.experimental.pallas import tpu as pltpu
from jax.experimental.pallas import tpu_sc as plsc

NC = 2
NS = 16
LN = 16
NW = NC * NS

SUB = 128
CHB = 2
CH = SUB * CHB
NBUF = 2


def _build(N, D, WD, WDP, PD, PV, per_w, n_chunks):
  mesh = plsc.VectorSubcoreMesh(
      core_axis_name="c", subcore_axis_name="s", num_cores=NC,
      num_subcores=NS)

  @functools.partial(
      pl.kernel,
      out_type=jax.ShapeDtypeStruct((N * D,), jnp.float32),
      mesh=mesh,
      compiler_params=pltpu.CompilerParams(
          needs_layout_passes=False, use_tc_tiling_on_sc=False),
      scratch_types=[
          [pltpu.VMEM((CHB, SUB), jnp.int32)] * NBUF,
          [pltpu.VMEM((CH,), jnp.int32)] * NBUF,
          [pltpu.VMEM((CH,), jnp.int32)] * NBUF,
          [pltpu.VMEM((CH, WDP), jnp.float32)] * NBUF,
          [pltpu.VMEM((CH * D,), jnp.float32)] * NBUF,
          pltpu.VMEM((PV * PD,), jnp.float32),
          pltpu.VMEM((PV * PD,), jnp.float32),
          [pltpu.SemaphoreType.DMA] * NBUF,
          [pltpu.SemaphoreType.DMA] * NBUF,
      ],
  )
  def k(wf_h, p1_h, p2_h, wt_h, p1t_h, p2t_h, out_h,
        widx, p1i, p2i, wrows, outv, p1t, p2t, gsem, wsem):
    wid = lax.axis_index("s") * NC + lax.axis_index("c")
    base = wid * per_w
    pltpu.sync_copy(p1t_h, p1t)
    pltpu.sync_copy(p2t_h, p2t)
    lane = lax.iota(jnp.int32, 16)

    def idx_dma(c, b):
      row0 = base + c * CH
      for bb in range(CHB):
        pltpu.sync_copy(wf_h.at[pl.ds(row0 + bb * SUB, SUB)], widx[b].at[bb])
      pltpu.sync_copy(p1_h.at[pl.ds(row0, CH)], p1i[b])
      pltpu.sync_copy(p2_h.at[pl.ds(row0, CH)], p2i[b])

    def fire_gathers(b):
      for bb in range(CHB):
        pltpu.async_copy(
            wt_h.at[widx[b].at[bb]],
            wrows[b].at[pl.ds(bb * SUB, SUB), :], gsem[b])

    def drain_gathers(b):
      pltpu.make_async_copy(
          wt_h.at[pl.ds(0, CH), :], wrows[b], gsem[b]).wait()

    def fire_writeout(c, b):
      row0 = base + c * CH
      pltpu.async_copy(outv[b], out_h.at[pl.ds(row0 * D, CH * D)], wsem[b])

    def drain_writeout(b):
      pltpu.make_async_copy(
          out_h.at[pl.ds(0, CH * D)], outv[b], wsem[b]).wait()

    def pos_compute(b):
      @pl.loop(0, CH // LN)
      def _pos(g):
        rowsD = (g * LN + lane) * D
        i1 = p1i[b][pl.ds(g * LN, LN)] * PD
        i2 = p2i[b][pl.ds(g * LN, LN)] * PD
        for j in range(PD):
          v1 = plsc.load_gather(p1t, [i1 + j])
          plsc.store_scatter(outv[b], [rowsD + (WD + j)], v1)
          v2 = plsc.load_gather(p2t, [i2 + j])
          plsc.store_scatter(outv[b], [rowsD + (WD + PD + j)], v2)

    def merge(b):
      @pl.loop(0, CH, unroll=2)
      def _merge(r):
        dst0 = r * D + lane
        for j in (0, LN, 2 * LN, WD - LN):
          v = wrows[b][r, pl.ds(j, LN)]
          plsc.store_scatter(outv[b], [dst0 + j], v)

    for b in range(NBUF):
      idx_dma(b, b)
      fire_gathers(b)

    @pl.loop(0, n_chunks, step=NBUF)
    def _outer(g):
      for b in range(NBUF):
        c = g + b

        @pl.when(g >= NBUF)
        def _():
          drain_writeout(b)

        pos_compute(b)
        drain_gathers(b)
        merge(b)
        fire_writeout(c, b)

        @pl.when(g < n_chunks - NBUF)
        def _():
          idx_dma(c + NBUF, b)
          fire_gathers(b)

    for b in range(NBUF):
      drain_writeout(b)

  return k


def kernel(word, pos1, pos2, word_table, pos1_table, pos2_table):
  B, L = word.shape
  V, WD = word_table.shape
  PV, PD = pos1_table.shape
  D = WD + 2 * PD
  N = B * L
  assert N % (NW * CH * NBUF) == 0
  per_w = N // NW
  n_chunks = per_w // CH

  wf = word.reshape(N).astype(jnp.int32)
  p1f = pos1.reshape(N).astype(jnp.int32)
  p2f = pos2.reshape(N).astype(jnp.int32)
  p1t = pos1_table.reshape(PV * PD)
  p2t = pos2_table.reshape(PV * PD)

  WDP = 64
  wt_pad = jnp.pad(word_table, ((0, 0), (0, WDP - WD)))

  k = _build(N, D, WD, WDP, PD, PV, per_w, n_chunks)
  out = k(wf, p1f, p2f, wt_pad, p1t, p2t)
  return out.reshape(B, L, D)


# speedup vs baseline: 12.3306x; 2.2961x over previous
import functools

import jax
import jax.numpy as jnp
from jax import lax
from jax.experimental import pallas as pl
from jax.experimental.pallas import tpu as pltpu
from jax.experimental.pallas import tpu_sc as plsc

NC = 2
NS = 16
LN = 16
NW = NC * NS

SUB = 128
CHB = 2
CH = SUB * CHB
NBUF = 3
PITCH = 128


def _build(N, D, WD, WDP, PD, PV, per_w, n_chunks):
  mesh = plsc.VectorSubcoreMesh(
      core_axis_name="c", subcore_axis_name="s", num_cores=NC,
      num_subcores=NS)

  @functools.partial(
      pl.kernel,
      out_type=jax.ShapeDtypeStruct((N, PITCH), jnp.float32),
      mesh=mesh,
      compiler_params=pltpu.CompilerParams(
          needs_layout_passes=False, use_tc_tiling_on_sc=False),
      scratch_types=[
          [pltpu.VMEM((CHB, SUB), jnp.int32)] * NBUF,
          [pltpu.VMEM((CH,), jnp.int32)] * NBUF,
          [pltpu.VMEM((CH,), jnp.int32)] * NBUF,
          [pltpu.VMEM((CH, PITCH), jnp.float32)] * NBUF,
          pltpu.VMEM((PV * PD,), jnp.float32),
          pltpu.VMEM((PV * PD,), jnp.float32),
          [pltpu.SemaphoreType.DMA] * NBUF,
          [pltpu.SemaphoreType.DMA] * NBUF,
      ],
  )
  def k(wf_h, p1_h, p2_h, wt_h, p1t_h, p2t_h, out_h,
        widx, p1i, p2i, outv, p1t, p2t, gsem, wsem):
    wid = lax.axis_index("s") * NC + lax.axis_index("c")
    base = wid * per_w
    pltpu.sync_copy(p1t_h, p1t)
    pltpu.sync_copy(p2t_h, p2t)
    lane = lax.iota(jnp.int32, 16)

    def idx_dma(c, b):
      row0 = base + c * CH
      for bb in range(CHB):
        pltpu.sync_copy(wf_h.at[pl.ds(row0 + bb * SUB, SUB)], widx[b].at[bb])
      pltpu.sync_copy(p1_h.at[pl.ds(row0, CH)], p1i[b])
      pltpu.sync_copy(p2_h.at[pl.ds(row0, CH)], p2i[b])

    def fire_gathers(b):
      for bb in range(CHB):
        pltpu.async_copy(
            wt_h.at[widx[b].at[bb]],
            outv[b].at[pl.ds(bb * SUB, SUB), :], gsem[b])

    def drain_gathers(b):
      pltpu.make_async_copy(
          wt_h.at[pl.ds(0, CH), :], outv[b], gsem[b]).wait()

    def fire_writeout(c, b):
      row0 = base + c * CH
      pltpu.async_copy(outv[b], out_h.at[pl.ds(row0, CH), :], wsem[b])

    def drain_writeout(b):
      pltpu.make_async_copy(
          out_h.at[pl.ds(0, CH), :], outv[b], wsem[b]).wait()

    def pos_compute(b):
      @pl.loop(0, CH // LN)
      def _pos(g):
        rows = g * LN + lane
        i1 = p1i[b][pl.ds(g * LN, LN)] * PD
        i2 = p2i[b][pl.ds(g * LN, LN)] * PD
        for j in range(PD):
          v1 = plsc.load_gather(p1t, [i1 + j])
          plsc.store_scatter(
              outv[b], [rows, jnp.full((LN,), WD + j, jnp.int32)], v1)
          v2 = plsc.load_gather(p2t, [i2 + j])
          plsc.store_scatter(
              outv[b], [rows, jnp.full((LN,), WD + PD + j, jnp.int32)], v2)

    for c0 in range(NBUF - 1):
      idx_dma(c0, c0)
      fire_gathers(c0)

    n_main = (n_chunks // NBUF) * NBUF

    def step(c, b):
      drain_gathers(b)
      pos_compute(b)
      fire_writeout(c, b)
      b2 = (b + NBUF - 1) % NBUF

      @pl.when(c + NBUF - 1 < n_chunks)
      def _():
        @pl.when(c >= 1)
        def _():
          drain_writeout(b2)

        idx_dma(c + NBUF - 1, b2)
        fire_gathers(b2)

    @pl.loop(0, n_main, step=NBUF)
    def _outer(g):
      for b in range(NBUF):
        step(g + b, b)

    for c in range(n_main, n_chunks):
      step(c, c % NBUF)

    for c in range(n_chunks - NBUF, n_chunks):
      drain_writeout(c % NBUF)

  return k


def kernel(word, pos1, pos2, word_table, pos1_table, pos2_table):
  B, L = word.shape
  V, WD = word_table.shape
  PV, PD = pos1_table.shape
  D = WD + 2 * PD
  N = B * L
  assert N % (NW * CH) == 0
  per_w = N // NW
  n_chunks = per_w // CH
  assert n_chunks >= NBUF

  wf = word.reshape(N).astype(jnp.int32)
  p1f = pos1.reshape(N).astype(jnp.int32)
  p2f = pos2.reshape(N).astype(jnp.int32)
  p1t = pos1_table.reshape(PV * PD)
  p2t = pos2_table.reshape(PV * PD)

  WDP = PITCH
  wt_pad = jnp.pad(word_table, ((0, 0), (0, WDP - WD)))

  k = _build(N, D, WD, WDP, PD, PV, per_w, n_chunks)
  out = k(wf, p1f, p2f, wt_pad, p1t, p2t)
  return out.reshape(B, L, PITCH)[:, :, :D]
